# Initial kernel scaffold; baseline (speedup 1.0000x reference)
import jax
import jax.numpy as jnp
from jax.experimental import pallas as pl


def kernel(pd_scores, pd_bboxes, anc_points, gt_labels, gt_bboxes, num_classes):
    raise NotImplementedError("write your pallas kernel here")


# speedup vs baseline: 26.2372x; 26.2372x over previous
import jax
import jax.numpy as jnp
from jax import lax
from jax.experimental import pallas as pl
from jax.experimental.pallas import tpu as pltpu

_IOU_WEIGHT = 3.0


def _body(ps_ref, pbT_ref, ancT_ref, gtl_ref, gtb_ref,
          labels_ref, bbox_ref, scores_ref, fg_ref):
    G = gtb_ref.shape[0]
    N = ps_ref.shape[0]
    C = ps_ref.shape[1]
    f32 = jnp.float32

    gx1 = gtb_ref[:, 0:1]
    gy1 = gtb_ref[:, 1:2]
    gx2 = gtb_ref[:, 2:3]
    gy2 = gtb_ref[:, 3:4]

    ax = ancT_ref[0:1, :]
    ay = ancT_ref[1:2, :]

    l = ax - gx1
    t = ay - gy1
    r = gx2 - ax
    b = gy2 - ay
    in_mask = jnp.minimum(jnp.minimum(l, t), jnp.minimum(r, b)) > 0.01

    px1 = pbT_ref[0:1, :]
    py1 = pbT_ref[1:2, :]
    px2 = pbT_ref[2:3, :]
    py2 = pbT_ref[3:4, :]

    ix1 = jnp.maximum(gx1, px1)
    iy1 = jnp.maximum(gy1, py1)
    ix2 = jnp.minimum(gx2, px2)
    iy2 = jnp.minimum(gy2, py2)
    inter = jnp.maximum(ix2 - ix1, 0.0) * jnp.maximum(iy2 - iy1, 0.0)
    area_g = (gx2 - gx1) * (gy2 - gy1)
    area_p = (px2 - px1) * (py2 - py1)
    union = area_g + area_p - inter + 1e-9
    iou = inter / union

    cls = gtl_ref[:, 0:1]
    onehot = (lax.broadcasted_iota(jnp.int32, (G, C), 1) == cls).astype(f32)
    ps = lax.dot_general(onehot, ps_ref[:, :],
                         (((1,), (1,)), ((), ())),
                         preferred_element_type=f32,
                         precision=lax.Precision.HIGHEST)

    cost_cls = jax.nn.softplus(-ps)
    cost_iou = -jnp.log(iou + 1e-8)
    not_in = jnp.where(in_mask, 0.0, 1.0)
    cost = cost_cls + _IOU_WEIGHT * cost_iou + 100000.0 * not_in

    masked_iou = jnp.where(in_mask, iou, 0.0)
    ks = jnp.maximum(
        jnp.sum(masked_iou, axis=1, keepdims=True).astype(jnp.int32), 1)
    ksf = ks.astype(f32)

    bits = lax.bitcast_convert_type(cost, jnp.int32)
    key = bits ^ ((bits >> 31) & jnp.int32(0x7FFFFFFF))

    lo = jnp.full((G, 1), jnp.iinfo(jnp.int32).min, jnp.int32)
    hi = jnp.full((G, 1), jnp.iinfo(jnp.int32).max, jnp.int32)
    for _ in range(32):
        mid = lo + lax.shift_right_logical(hi - lo, 1)
        cnt = jnp.sum((key <= mid).astype(f32), axis=1, keepdims=True)
        ge = cnt >= ksf
        hi = jnp.where(ge, mid, hi)
        lo = jnp.where(ge, lo, mid + 1)
    thr = lo

    less = key < thr
    eq = key == thr
    cl = jnp.sum(less.astype(f32), axis=1, keepdims=True)
    col = lax.broadcasted_iota(jnp.int32, (1, N), 1)

    li = jnp.zeros((G, 1), jnp.int32)
    hi2 = jnp.full((G, 1), N, jnp.int32)
    for _ in range(14):
        mid = (li + hi2) >> 1
        cnt = cl + jnp.sum((eq & (col < mid)).astype(f32), axis=1,
                           keepdims=True)
        ge = cnt >= ksf
        hi2 = jnp.where(ge, mid, hi2)
        li = jnp.where(ge, li, mid + 1)
    m0 = (less | (eq & (col < hi2))).astype(f32)

    colsum = jnp.sum(m0, axis=0, keepdims=True)
    mult = colsum > 1.0
    minc = jnp.min(cost, axis=0, keepdims=True)
    rowi = lax.broadcasted_iota(jnp.int32, (G, N), 0).astype(f32)
    am = jnp.min(jnp.where(cost == minc, rowi, f32(G)), axis=0,
                 keepdims=True)
    m = jnp.where(mult, (rowi == am).astype(f32), m0)

    fg = jnp.sum(m, axis=0, keepdims=True) > 0.0

    labels_f = jnp.sum(m * cls.astype(f32), axis=0, keepdims=True)
    labels_ref[:, :] = jnp.where(fg, labels_f.astype(jnp.int32), C)
    fg_ref[:, :] = fg.astype(jnp.int32)

    w = m * masked_iou
    scores_ref[:, :] = lax.dot_general(
        w, onehot, (((0,), (0,)), ((), ())),
        preferred_element_type=f32, precision=lax.Precision.HIGHEST)
    bbox_ref[:, :] = lax.dot_general(
        m, gtb_ref[:, :], (((0,), (0,)), ((), ())),
        preferred_element_type=f32, precision=lax.Precision.HIGHEST)


def kernel(pd_scores, pd_bboxes, anc_points, gt_labels, gt_bboxes,
           num_classes):
    del num_classes
    N, C = pd_scores.shape
    G = gt_bboxes.shape[0]
    gtl = gt_labels.astype(jnp.int32)

    labels2, bboxes, scores, fg2 = pl.pallas_call(
        _body,
        out_shape=[
            jax.ShapeDtypeStruct((1, N), jnp.int32),
            jax.ShapeDtypeStruct((N, 4), jnp.float32),
            jax.ShapeDtypeStruct((N, C), jnp.float32),
            jax.ShapeDtypeStruct((1, N), jnp.int32),
        ],
    )(pd_scores, pd_bboxes.T, anc_points.T, gtl, gt_bboxes)

    return (labels2.reshape(N), bboxes, scores,
            fg2.reshape(N).astype(jnp.bool_))


# speedup vs baseline: 27.1811x; 1.0360x over previous
import jax
import jax.numpy as jnp
from jax import lax
from jax.experimental import pallas as pl
from jax.experimental.pallas import tpu as pltpu

_IOU_WEIGHT = 3.0


def _body(ps_ref, pbT_ref, ancT_ref, gtl_ref, gtb_ref,
          labels_ref, bbox_ref, scores_ref, fg_ref):
    G = gtb_ref.shape[0]
    N = ps_ref.shape[0]
    C = ps_ref.shape[1]
    f32 = jnp.float32

    gx1 = gtb_ref[:, 0:1]
    gy1 = gtb_ref[:, 1:2]
    gx2 = gtb_ref[:, 2:3]
    gy2 = gtb_ref[:, 3:4]

    ax = ancT_ref[0:1, :]
    ay = ancT_ref[1:2, :]

    l = ax - gx1
    t = ay - gy1
    r = gx2 - ax
    b = gy2 - ay
    in_mask = jnp.minimum(jnp.minimum(l, t), jnp.minimum(r, b)) > 0.01

    px1 = pbT_ref[0:1, :]
    py1 = pbT_ref[1:2, :]
    px2 = pbT_ref[2:3, :]
    py2 = pbT_ref[3:4, :]

    ix1 = jnp.maximum(gx1, px1)
    iy1 = jnp.maximum(gy1, py1)
    ix2 = jnp.minimum(gx2, px2)
    iy2 = jnp.minimum(gy2, py2)
    inter = jnp.maximum(ix2 - ix1, 0.0) * jnp.maximum(iy2 - iy1, 0.0)
    area_g = (gx2 - gx1) * (gy2 - gy1)
    area_p = (px2 - px1) * (py2 - py1)
    union = area_g + area_p - inter + 1e-9
    iou = inter / union

    cls = gtl_ref[:, 0:1]
    onehot = (lax.broadcasted_iota(jnp.int32, (G, C), 1) == cls).astype(f32)
    ps = lax.dot_general(onehot, ps_ref[:, :],
                         (((1,), (1,)), ((), ())),
                         preferred_element_type=f32,
                         precision=lax.Precision.HIGHEST)

    cost_cls = jax.nn.softplus(-ps)
    cost_iou = -jnp.log(iou + 1e-8)
    not_in = jnp.where(in_mask, 0.0, 1.0)
    cost = cost_cls + _IOU_WEIGHT * cost_iou + 100000.0 * not_in

    masked_iou = jnp.where(in_mask, iou, 0.0)
    ks = jnp.maximum(
        jnp.sum(masked_iou, axis=1, keepdims=True).astype(jnp.int32), 1)
    ksf = ks.astype(f32)

    bits = lax.bitcast_convert_type(cost, jnp.int32)
    key = bits ^ ((bits >> 31) & jnp.int32(0x7FFFFFFF))

    lo = jnp.min(key, axis=1, keepdims=True)
    hi = jnp.max(key, axis=1, keepdims=True)

    def _val_cond(carry):
        lo_c, hi_c = carry
        return jnp.any(lo_c < hi_c)

    def _val_body(carry):
        lo_c, hi_c = carry
        mid = lo_c + lax.shift_right_logical(hi_c - lo_c, 1)
        cnt = jnp.sum((key <= mid).astype(f32), axis=1, keepdims=True)
        ge = cnt >= ksf
        return jnp.where(ge, lo_c, mid + 1), jnp.where(ge, mid, hi_c)

    thr, _ = lax.while_loop(_val_cond, _val_body, (lo, hi))

    less = key < thr
    eq = key == thr
    cl = jnp.sum(less.astype(f32), axis=1, keepdims=True)
    cle = jnp.sum((key <= thr).astype(f32), axis=1, keepdims=True)
    col = lax.broadcasted_iota(jnp.int32, (1, N), 1)

    def _tie_search(_):
        li = jnp.zeros((G, 1), jnp.int32)
        hi2 = jnp.full((G, 1), N, jnp.int32)
        for _ in range(14):
            mid = (li + hi2) >> 1
            cnt = cl + jnp.sum((eq & (col < mid)).astype(f32), axis=1,
                               keepdims=True)
            ge = cnt >= ksf
            hi2 = jnp.where(ge, mid, hi2)
            li = jnp.where(ge, li, mid + 1)
        return (less | (eq & (col < hi2))).astype(f32)

    def _take_all(_):
        return (less | eq).astype(f32)

    m0 = lax.cond(jnp.any(cle > ksf), _tie_search, _take_all, 0)

    colsum = jnp.sum(m0, axis=0, keepdims=True)
    mult = colsum > 1.0
    minc = jnp.min(cost, axis=0, keepdims=True)
    rowi = lax.broadcasted_iota(jnp.int32, (G, N), 0).astype(f32)
    am = jnp.min(jnp.where(cost == minc, rowi, f32(G)), axis=0,
                 keepdims=True)
    m = jnp.where(mult, (rowi == am).astype(f32), m0)

    fg = jnp.sum(m, axis=0, keepdims=True) > 0.0

    labels_f = jnp.sum(m * cls.astype(f32), axis=0, keepdims=True)
    labels_ref[:, :] = jnp.where(fg, labels_f.astype(jnp.int32), C)
    fg_ref[:, :] = fg.astype(jnp.int32)

    w = m * masked_iou
    scores_ref[:, :] = lax.dot_general(
        w, onehot, (((0,), (0,)), ((), ())),
        preferred_element_type=f32, precision=lax.Precision.HIGHEST)
    bbox_ref[:, :] = lax.dot_general(
        m, gtb_ref[:, :], (((0,), (0,)), ((), ())),
        preferred_element_type=f32, precision=lax.Precision.HIGHEST)


def kernel(pd_scores, pd_bboxes, anc_points, gt_labels, gt_bboxes,
           num_classes):
    del num_classes
    N, C = pd_scores.shape
    G = gt_bboxes.shape[0]
    gtl = gt_labels.astype(jnp.int32)

    labels2, bboxes, scores, fg2 = pl.pallas_call(
        _body,
        out_shape=[
            jax.ShapeDtypeStruct((1, N), jnp.int32),
            jax.ShapeDtypeStruct((N, 4), jnp.float32),
            jax.ShapeDtypeStruct((N, C), jnp.float32),
            jax.ShapeDtypeStruct((1, N), jnp.int32),
        ],
    )(pd_scores, pd_bboxes.T, anc_points.T, gtl, gt_bboxes)

    return (labels2.reshape(N), bboxes, scores,
            fg2.reshape(N).astype(jnp.bool_))
